# baseline (speedup 1.0000x reference)
import jax
import jax.numpy as jnp
from jax.experimental import pallas as pl

_B, _N, _C = 128, 1024, 96
_NCLS = 1000
_K = 256
_BB = 8


def _select_weights(r):
    kk = jnp.int32(_K)
    ib = jax.lax.bitcast_convert_type(r, jnp.int32)
    key = jnp.where(ib < 0, ib ^ jnp.int32(0x7FFFFFFF), ib)

    cnt0 = jnp.sum((key >= 0).astype(jnp.int32), axis=1, keepdims=True)
    prefix = jnp.where(cnt0 >= kk, jnp.int32(0), jnp.int32(-2147483648))

    def step(j, p):
        bit = jnp.int32(1) << (jnp.int32(30) - j)
        cand = p | bit
        cnt = jnp.sum((key >= cand).astype(jnp.int32), axis=1, keepdims=True)
        return jnp.where(cnt >= kk, cand, p)

    t = jax.lax.fori_loop(0, 31, step, prefix)

    gt = key > t
    tie = key == t
    n_gt = jnp.sum(gt.astype(jnp.int32), axis=1, keepdims=True)
    need = kk - n_gt

    idx = jax.lax.broadcasted_iota(jnp.int32, r.shape, 1)
    key2 = jnp.where(tie, jnp.int32(_N - 1) - idx, jnp.int32(-1))
    p2 = jnp.zeros_like(need)

    def step2(j, p):
        bit = jnp.int32(1) << (jnp.int32(9) - j)
        cand = p | bit
        cnt = jnp.sum((key2 >= cand).astype(jnp.int32), axis=1, keepdims=True)
        return jnp.where(cnt >= need, cand, p)

    p2 = jax.lax.fori_loop(0, 10, step2, p2)
    sel = gt | (key2 >= p2)

    m = jnp.max(r, axis=1, keepdims=True)
    e = jnp.where(sel, jnp.exp(r - m), jnp.float32(0))
    z = jnp.sum(e, axis=1, keepdims=True)
    return e / z


def _body(rel_ref, tok_ref, fcw_ref, fcb_ref, out_ref):
    w = _select_weights(rel_ref[...])
    tok = tok_ref[...]
    feat = jnp.sum(tok * w[:, :, None], axis=1)
    logits = jax.lax.dot_general(
        feat, fcw_ref[...], (((1,), (1,)), ((), ())),
        preferred_element_type=jnp.float32)
    out_ref[...] = logits + fcb_ref[...]


def kernel(tokens, reliability, fc_w, fc_b):
    fcb2 = fc_b.reshape(1, _NCLS)
    return pl.pallas_call(
        _body,
        grid=(_B // _BB,),
        in_specs=[
            pl.BlockSpec((_BB, _N), lambda i: (i, 0)),
            pl.BlockSpec((_BB, _N, _C), lambda i: (i, 0, 0)),
            pl.BlockSpec((_NCLS, _C), lambda i: (0, 0)),
            pl.BlockSpec((1, _NCLS), lambda i: (0, 0)),
        ],
        out_specs=pl.BlockSpec((_BB, _NCLS), lambda i: (i, 0)),
        out_shape=jax.ShapeDtypeStruct((_B, _NCLS), jnp.float32),
    )(reliability, tokens, fc_w, fcb2)
